# baseline (device time: 32612 ns/iter reference)
import jax
import jax.numpy as jnp
from jax import lax
from jax.experimental import pallas as pl
from jax.experimental.pallas import tpu as pltpu

NC = 16
DY = (0, 1, 2, 3, 4)
DX = (5, 6, 7, 8, 9)
DZ = (10, 11, 12, 13, 14, 15)
Y_ORDER = (5, 10, 6, 11, 7, 12, 8, 13, 9, 14, 15, 0, 1, 2, 3, 4)


def kernel(x):
    m, n = x.shape
    n_out = n // 2
    qm = m // 4
    ck = qm // NC

    def body(x_ref, out_ref, local_sem,
             ysend, yrecv, xqs, xqr, zqs, zqr, xrs, xrr, zrs, zrr):
        my_x = lax.axis_index("x")
        my_y = lax.axis_index("y")
        my_z = lax.axis_index("z")
        ypeer = (my_x, 1 - my_y, my_z)
        xsib = (1 - my_x, my_y, my_z)
        zsib = (my_x, my_y, 1 - my_z)

        qoff = my_x * (2 * qm) + my_z * qm
        doff = (1 - my_x) * (2 * qm) + (1 - my_z) * qm
        recvbase = (1 - my_y) * m
        row_q = recvbase + qoff
        row_xq = recvbase + (1 - my_x) * (2 * qm) + my_z * qm
        row_zq = recvbase + my_x * (2 * qm) + (1 - my_z) * qm

        local_copy = pltpu.make_async_copy(
            x_ref.at[:, pl.ds(my_y * n_out, n_out)],
            out_ref.at[pl.ds(my_y * m, m), :],
            local_sem,
        )
        local_copy.start()

        barrier_sem = pltpu.get_barrier_semaphore()
        for p in (ypeer, xsib, zsib):
            pl.semaphore_signal(
                barrier_sem, inc=1,
                device_id=p, device_id_type=pl.DeviceIdType.MESH,
            )
        pl.semaphore_wait(barrier_sem, 3)

        def y_send(b_row, dst_row, sem_idx):
            return pltpu.make_async_remote_copy(
                src_ref=x_ref.at[
                    pl.ds(b_row, ck), pl.ds((1 - my_y) * n_out, n_out)
                ],
                dst_ref=out_ref.at[pl.ds(dst_row, ck), :],
                send_sem=ysend.at[sem_idx],
                recv_sem=yrecv.at[sem_idx],
                device_id=ypeer,
                device_id_type=pl.DeviceIdType.MESH,
            )

        def exch(row, ssem, rsem, peer):
            return pltpu.make_async_remote_copy(
                src_ref=out_ref.at[pl.ds(row, ck), :],
                dst_ref=out_ref.at[pl.ds(row, ck), :],
                send_sem=ssem,
                recv_sem=rsem,
                device_id=peer,
                device_id_type=pl.DeviceIdType.MESH,
            )

        y_rd, yd_rd = {}, {}
        for c in Y_ORDER:
            y_rd[c] = y_send(qoff + c * ck, my_y * m + qoff + c * ck, c)
            y_rd[c].start()
        for i, c in enumerate(DY):
            yd_rd[c] = y_send(doff + c * ck, my_y * m + doff + c * ck, NC + i)
            yd_rd[c].start()

        xq_rd, zq_rd = {}, {}
        for c in Y_ORDER:
            y_rd[c].wait_recv()
            xq_rd[c] = exch(row_q + c * ck, xqs.at[c], xqr.at[c], xsib)
            xq_rd[c].start()
            zq_rd[c] = exch(row_q + c * ck, zqs.at[c], zqr.at[c], zsib)
            zq_rd[c].start()

        xr_rd, zr_rd = {}, {}
        for i in range(max(len(DX), len(DZ))):
            if i < len(DX):
                cx = DX[i]
                zq_rd[cx].wait_recv()
                xr_rd[cx] = exch(
                    row_zq + cx * ck, xrs.at[cx], xrr.at[cx], xsib
                )
                xr_rd[cx].start()
            if i < len(DZ):
                cz = DZ[i]
                xq_rd[cz].wait_recv()
                zr_rd[cz] = exch(
                    row_xq + cz * ck, zrs.at[cz], zrr.at[cz], zsib
                )
                zr_rd[cz].start()

        for c in DY:
            yd_rd[c].wait_recv()
        for c in Y_ORDER:
            if c not in DX:
                zq_rd[c].wait_recv()
            if c not in DZ:
                xq_rd[c].wait_recv()
        for c in DX:
            xr_rd[c].wait_recv()
        for c in DZ:
            zr_rd[c].wait_recv()
        for c in range(NC):
            y_rd[c].wait_send()
            xq_rd[c].wait_send()
            zq_rd[c].wait_send()
        for c in DY:
            yd_rd[c].wait_send()
        for c in DX:
            xr_rd[c].wait_send()
        for c in DZ:
            zr_rd[c].wait_send()
        local_copy.wait()

    return pl.pallas_call(
        body,
        out_shape=jax.ShapeDtypeStruct((2 * m, n_out), x.dtype),
        in_specs=[pl.BlockSpec(memory_space=pl.ANY)],
        out_specs=pl.BlockSpec(memory_space=pl.ANY),
        scratch_shapes=[
            pltpu.SemaphoreType.DMA,
            pltpu.SemaphoreType.DMA((NC + len(DY),)),
            pltpu.SemaphoreType.DMA((NC + len(DY),)),
            pltpu.SemaphoreType.DMA((NC,)),
            pltpu.SemaphoreType.DMA((NC,)),
            pltpu.SemaphoreType.DMA((NC,)),
            pltpu.SemaphoreType.DMA((NC,)),
            pltpu.SemaphoreType.DMA((NC,)),
            pltpu.SemaphoreType.DMA((NC,)),
            pltpu.SemaphoreType.DMA((NC,)),
            pltpu.SemaphoreType.DMA((NC,)),
        ],
        compiler_params=pltpu.CompilerParams(collective_id=0),
    )(x)


# device time: 32509 ns/iter; 1.0032x vs baseline; 1.0032x over previous
import jax
import jax.numpy as jnp
from jax import lax
from jax.experimental import pallas as pl
from jax.experimental.pallas import tpu as pltpu

NC = 8
DY = (0, 1)
DX = (2, 3, 4)
DZ = (5, 6, 7)
Y_ORDER = (2, 5, 3, 6, 4, 7, 0, 1)


def kernel(x):
    m, n = x.shape
    n_out = n // 2
    qm = m // 4
    ck = qm // NC

    def body(x_ref, out_ref, local_sem,
             ysend, yrecv, xqs, xqr, zqs, zqr, xrs, xrr, zrs, zrr):
        my_x = lax.axis_index("x")
        my_y = lax.axis_index("y")
        my_z = lax.axis_index("z")
        ypeer = (my_x, 1 - my_y, my_z)
        xsib = (1 - my_x, my_y, my_z)
        zsib = (my_x, my_y, 1 - my_z)

        qoff = my_x * (2 * qm) + my_z * qm
        doff = (1 - my_x) * (2 * qm) + (1 - my_z) * qm
        recvbase = (1 - my_y) * m
        row_q = recvbase + qoff
        row_xq = recvbase + (1 - my_x) * (2 * qm) + my_z * qm
        row_zq = recvbase + my_x * (2 * qm) + (1 - my_z) * qm

        local_copy = pltpu.make_async_copy(
            x_ref.at[:, pl.ds(my_y * n_out, n_out)],
            out_ref.at[pl.ds(my_y * m, m), :],
            local_sem,
        )
        local_copy.start()

        barrier_sem = pltpu.get_barrier_semaphore()
        for p in (ypeer, xsib, zsib):
            pl.semaphore_signal(
                barrier_sem, inc=1,
                device_id=p, device_id_type=pl.DeviceIdType.MESH,
            )
        pl.semaphore_wait(barrier_sem, 3)

        def y_send(b_row, dst_row, sem_idx):
            return pltpu.make_async_remote_copy(
                src_ref=x_ref.at[
                    pl.ds(b_row, ck), pl.ds((1 - my_y) * n_out, n_out)
                ],
                dst_ref=out_ref.at[pl.ds(dst_row, ck), :],
                send_sem=ysend.at[sem_idx],
                recv_sem=yrecv.at[sem_idx],
                device_id=ypeer,
                device_id_type=pl.DeviceIdType.MESH,
            )

        def exch(row, ssem, rsem, peer):
            return pltpu.make_async_remote_copy(
                src_ref=out_ref.at[pl.ds(row, ck), :],
                dst_ref=out_ref.at[pl.ds(row, ck), :],
                send_sem=ssem,
                recv_sem=rsem,
                device_id=peer,
                device_id_type=pl.DeviceIdType.MESH,
            )

        y_rd, yd_rd = {}, {}
        for c in Y_ORDER:
            y_rd[c] = y_send(qoff + c * ck, my_y * m + qoff + c * ck, c)
            y_rd[c].start()
        for i, c in enumerate(DY):
            yd_rd[c] = y_send(doff + c * ck, my_y * m + doff + c * ck, NC + i)
            yd_rd[c].start()

        xq_rd, zq_rd = {}, {}
        for c in Y_ORDER:
            y_rd[c].wait_recv()
            xq_rd[c] = exch(row_q + c * ck, xqs.at[c], xqr.at[c], xsib)
            xq_rd[c].start()
            zq_rd[c] = exch(row_q + c * ck, zqs.at[c], zqr.at[c], zsib)
            zq_rd[c].start()

        xr_rd, zr_rd = {}, {}
        for cx, cz in zip(DX, DZ):
            zq_rd[cx].wait_recv()
            xr_rd[cx] = exch(row_zq + cx * ck, xrs.at[cx], xrr.at[cx], xsib)
            xr_rd[cx].start()
            xq_rd[cz].wait_recv()
            zr_rd[cz] = exch(row_xq + cz * ck, zrs.at[cz], zrr.at[cz], zsib)
            zr_rd[cz].start()

        for c in DY:
            yd_rd[c].wait_recv()
        for c in Y_ORDER:
            if c not in DX:
                zq_rd[c].wait_recv()
            if c not in DZ:
                xq_rd[c].wait_recv()
        for c in DX:
            xr_rd[c].wait_recv()
        for c in DZ:
            zr_rd[c].wait_recv()
        for c in range(NC):
            y_rd[c].wait_send()
            xq_rd[c].wait_send()
            zq_rd[c].wait_send()
        for c in DY:
            yd_rd[c].wait_send()
        for c in DX:
            xr_rd[c].wait_send()
        for c in DZ:
            zr_rd[c].wait_send()
        local_copy.wait()

    return pl.pallas_call(
        body,
        out_shape=jax.ShapeDtypeStruct((2 * m, n_out), x.dtype),
        in_specs=[pl.BlockSpec(memory_space=pl.ANY)],
        out_specs=pl.BlockSpec(memory_space=pl.ANY),
        scratch_shapes=[
            pltpu.SemaphoreType.DMA,
            pltpu.SemaphoreType.DMA((NC + len(DY),)),
            pltpu.SemaphoreType.DMA((NC + len(DY),)),
            pltpu.SemaphoreType.DMA((NC,)),
            pltpu.SemaphoreType.DMA((NC,)),
            pltpu.SemaphoreType.DMA((NC,)),
            pltpu.SemaphoreType.DMA((NC,)),
            pltpu.SemaphoreType.DMA((NC,)),
            pltpu.SemaphoreType.DMA((NC,)),
            pltpu.SemaphoreType.DMA((NC,)),
            pltpu.SemaphoreType.DMA((NC,)),
        ],
        compiler_params=pltpu.CompilerParams(collective_id=0),
    )(x)
